# baseline (speedup 1.0000x reference)
import jax
import jax.numpy as jnp
from jax import lax
from jax.experimental import pallas as pl
from jax.experimental.pallas import tpu as pltpu

_NT = 8


def _sts_kernel(x_hbm, lbl_hbm, w1_ref, b1_ref, w2_ref, b2_ref,
                emb_ref, loss_ref,
                xs, lbls, xsem, lsem):
    B, T, D = x_hbm.shape
    tt = B // _NT
    inv_t = 1.0 / T
    for c in range(_NT):
        pltpu.make_async_copy(
            x_hbm.at[pl.ds(c * tt, tt)], xs.at[c], xsem.at[c]).start()
        n = (c + 1) * tt
        pltpu.make_async_copy(
            lbl_hbm.at[pl.ds(0, n), pl.ds(c * tt, tt)],
            lbls.at[pl.ds(0, n), pl.ds(c * tt, tt)], lsem.at[c]).start()

    w1 = w1_ref[...]
    b1 = b1_ref[...]
    w2 = w2_ref[...]
    b2 = b2_ref[...]

    es = []
    sqs = []
    acc = jnp.zeros((1, tt), dtype=jnp.float32)
    for c in range(_NT):
        pltpu.make_async_copy(xs.at[c], xs.at[c], xsem.at[c]).wait()
        pooled = jnp.sum(xs[c], axis=1) * inv_t
        h = jnp.maximum(
            jnp.dot(pooled, w1, preferred_element_type=jnp.float32) + b1, 0.0)
        e = jnp.dot(h, w2, preferred_element_type=jnp.float32) + b2
        emb_ref[pl.ds(c * tt, tt), :] = e
        es.append(e)
        sqs.append(jnp.sum(e * e, axis=1, keepdims=True))

        pltpu.make_async_copy(
            lbls.at[pl.ds(0, (c + 1) * tt), pl.ds(c * tt, tt)],
            lbls.at[pl.ds(0, (c + 1) * tt), pl.ds(c * tt, tt)],
            lsem.at[c]).wait()
        sq_col = jnp.transpose(sqs[c])
        for i in range(c + 1):
            gram = lax.dot_general(
                es[i], e, dimension_numbers=(((1,), (1,)), ((), ())),
                preferred_element_type=jnp.float32)
            d2 = jnp.maximum(sqs[i] + sq_col - 2.0 * gram, 0.0)
            diff = jnp.sqrt(d2) - lbls[pl.ds(i * tt, tt), pl.ds(c * tt, tt)]
            if i == c:
                row = lax.broadcasted_iota(jnp.int32, (tt, tt), 0)
                col = lax.broadcasted_iota(jnp.int32, (tt, tt), 1)
                se = jnp.where(col > row, diff * diff, 0.0)
            else:
                se = diff * diff
            acc = acc + jnp.sum(se, axis=0, keepdims=True)

    inv_active = 1.0 / float(B * (B - 1) // 2)
    loss_ref[0, 0] = jnp.sum(acc) * inv_active


def kernel(x, labels, w1, b1, w2, b2):
    B, T, D = x.shape
    H = w1.shape[1]
    tt = B // _NT
    anyspec = pl.BlockSpec(memory_space=pl.ANY)
    vmem = pl.BlockSpec(memory_space=pltpu.MemorySpace.VMEM)
    smem = pl.BlockSpec(memory_space=pltpu.MemorySpace.SMEM)
    emb, loss = pl.pallas_call(
        _sts_kernel,
        out_shape=(jax.ShapeDtypeStruct((B, H), jnp.float32),
                   jax.ShapeDtypeStruct((1, 1), jnp.float32)),
        in_specs=[anyspec, anyspec, vmem, vmem, vmem, vmem],
        out_specs=(vmem, smem),
        scratch_shapes=[
            pltpu.VMEM((_NT, tt, T, D), jnp.float32),
            pltpu.VMEM((B, B), jnp.float32),
            pltpu.SemaphoreType.DMA((_NT,)),
            pltpu.SemaphoreType.DMA((_NT,)),
        ],
    )(x, labels, w1, b1, w2, b2)
    return emb, loss[0, 0]


# speedup vs baseline: 1.0784x; 1.0784x over previous
import jax
import jax.numpy as jnp
from jax import lax
from jax.experimental import pallas as pl
from jax.experimental.pallas import tpu as pltpu

_NT = 4


def _sts_kernel(x_hbm, lbl_hbm, w1_ref, b1_ref, w2_ref, b2_ref,
                emb_ref, loss_ref,
                xs, lbls, xsem, lsem):
    B, T, D = x_hbm.shape
    tt = B // _NT
    inv_t = 1.0 / T
    for c in range(_NT):
        pltpu.make_async_copy(
            x_hbm.at[pl.ds(c * tt, tt)], xs.at[c], xsem.at[c]).start()
        n = (c + 1) * tt
        pltpu.make_async_copy(
            lbl_hbm.at[pl.ds(0, n), pl.ds(c * tt, tt)],
            lbls.at[pl.ds(0, n), pl.ds(c * tt, tt)], lsem.at[c]).start()

    w1 = w1_ref[...]
    b1 = b1_ref[...]
    w2 = w2_ref[...]
    b2 = b2_ref[...]

    es = []
    sqs = []
    acc = jnp.zeros((1, tt), dtype=jnp.float32)
    for c in range(_NT):
        pltpu.make_async_copy(xs.at[c], xs.at[c], xsem.at[c]).wait()
        pooled = jnp.sum(xs[c], axis=1) * inv_t
        h = jnp.maximum(
            jnp.dot(pooled, w1, preferred_element_type=jnp.float32) + b1, 0.0)
        e = jnp.dot(h, w2, preferred_element_type=jnp.float32) + b2
        emb_ref[pl.ds(c * tt, tt), :] = e
        es.append(e)
        sqs.append(jnp.sum(e * e, axis=1, keepdims=True))

        pltpu.make_async_copy(
            lbls.at[pl.ds(0, (c + 1) * tt), pl.ds(c * tt, tt)],
            lbls.at[pl.ds(0, (c + 1) * tt), pl.ds(c * tt, tt)],
            lsem.at[c]).wait()
        sq_col = jnp.transpose(sqs[c])
        for i in range(c + 1):
            gram = lax.dot_general(
                es[i], e, dimension_numbers=(((1,), (1,)), ((), ())),
                preferred_element_type=jnp.float32)
            d2 = jnp.maximum(sqs[i] + sq_col - 2.0 * gram, 0.0)
            diff = jnp.sqrt(d2) - lbls[pl.ds(i * tt, tt), pl.ds(c * tt, tt)]
            if i == c:
                row = lax.broadcasted_iota(jnp.int32, (tt, tt), 0)
                col = lax.broadcasted_iota(jnp.int32, (tt, tt), 1)
                se = jnp.where(col > row, diff * diff, 0.0)
            else:
                se = diff * diff
            acc = acc + jnp.sum(se, axis=0, keepdims=True)

    inv_active = 1.0 / float(B * (B - 1) // 2)
    loss_ref[0, 0] = jnp.sum(acc) * inv_active


def kernel(x, labels, w1, b1, w2, b2):
    B, T, D = x.shape
    H = w1.shape[1]
    tt = B // _NT
    anyspec = pl.BlockSpec(memory_space=pl.ANY)
    vmem = pl.BlockSpec(memory_space=pltpu.MemorySpace.VMEM)
    smem = pl.BlockSpec(memory_space=pltpu.MemorySpace.SMEM)
    emb, loss = pl.pallas_call(
        _sts_kernel,
        out_shape=(jax.ShapeDtypeStruct((B, H), jnp.float32),
                   jax.ShapeDtypeStruct((1, 1), jnp.float32)),
        in_specs=[anyspec, anyspec, vmem, vmem, vmem, vmem],
        out_specs=(vmem, smem),
        scratch_shapes=[
            pltpu.VMEM((_NT, tt, T, D), jnp.float32),
            pltpu.VMEM((B, B), jnp.float32),
            pltpu.SemaphoreType.DMA((_NT,)),
            pltpu.SemaphoreType.DMA((_NT,)),
        ],
    )(x, labels, w1, b1, w2, b2)
    return emb, loss[0, 0]
